# Initial kernel scaffold; baseline (speedup 1.0000x reference)
import jax
import jax.numpy as jnp
from jax.experimental import pallas as pl


def kernel(x, edge_attr, cell_ef, params, edge_index):
    raise NotImplementedError("write your pallas kernel here")


# speedup vs baseline: 1.0091x; 1.0091x over previous
import functools
import numpy as np
import jax
import jax.numpy as jnp
from jax.experimental import pallas as pl

N = 10000
E = 320000
H = 64
H1 = 32
H2 = 32
HEADS = 8
DH = 8

NBLK = 2000
EBLK = 4000


def _leaky(v):
    return jax.nn.leaky_relu(v, 0.2)


def _k_dimup(x_ref, w_ref, b_ref, o_ref):
    o_ref[...] = _leaky(jnp.dot(x_ref[...], w_ref[...]) + b_ref[...])


def _dimup(x, w, b):
    return pl.pallas_call(
        _k_dimup,
        grid=(N // NBLK,),
        in_specs=[
            pl.BlockSpec((NBLK, 128), lambda i: (i, 0)),
            pl.BlockSpec((128, H), lambda i: (0, 0)),
            pl.BlockSpec((H,), lambda i: (0,)),
        ],
        out_specs=pl.BlockSpec((NBLK, H), lambda i: (i, 0)),
        out_shape=jax.ShapeDtypeStruct((N, H), jnp.float32),
    )(x, w, b)


def _k_msg(gs_ref, gd_ref, ea_ref, w1_ref, b1_ref, w2_ref, b2_ref, m_ref):
    m_in = jnp.concatenate([gs_ref[...], gd_ref[...], ea_ref[...]], axis=1)
    hm = _leaky(jnp.dot(m_in, w1_ref[...]) + b1_ref[...])
    m_ref[...] = jnp.dot(hm, w2_ref[...]) + b2_ref[...]


def _msg(gs, gd, ea, w1, b1, w2, b2):
    return pl.pallas_call(
        _k_msg,
        grid=(E // EBLK,),
        in_specs=[
            pl.BlockSpec((EBLK, H), lambda i: (i, 0)),
            pl.BlockSpec((EBLK, H), lambda i: (i, 0)),
            pl.BlockSpec((EBLK, 2), lambda i: (i, 0)),
            pl.BlockSpec((2 * H + 2, H), lambda i: (0, 0)),
            pl.BlockSpec((H,), lambda i: (0,)),
            pl.BlockSpec((H, 1 + H1 + H2), lambda i: (0, 0)),
            pl.BlockSpec((1 + H1 + H2,), lambda i: (0,)),
        ],
        out_specs=pl.BlockSpec((EBLK, 1 + H1 + H2), lambda i: (i, 0)),
        out_shape=jax.ShapeDtypeStruct((E, 1 + H1 + H2), jnp.float32),
    )(gs, gd, ea, w1, b1, w2, b2)


def _k_readout(h_ref, s_ref, mx_ref, w1_ref, b1_ref, w2_ref, b2_ref, o_ref):
    r = jnp.concatenate([h_ref[...], s_ref[...], mx_ref[...]], axis=1)
    hr = _leaky(jnp.dot(r, w1_ref[...]) + b1_ref[...])
    o_ref[...] = jnp.dot(hr, w2_ref[...]) + b2_ref[...] + h_ref[...]


def _readout(h, s, mx, w1, b1, w2, b2):
    return pl.pallas_call(
        _k_readout,
        grid=(N // NBLK,),
        in_specs=[
            pl.BlockSpec((NBLK, H), lambda i: (i, 0)),
            pl.BlockSpec((NBLK, H1), lambda i: (i, 0)),
            pl.BlockSpec((NBLK, H2), lambda i: (i, 0)),
            pl.BlockSpec((H + H1 + H2, H), lambda i: (0, 0)),
            pl.BlockSpec((H,), lambda i: (0,)),
            pl.BlockSpec((H, H), lambda i: (0, 0)),
            pl.BlockSpec((H,), lambda i: (0,)),
        ],
        out_specs=pl.BlockSpec((NBLK, H), lambda i: (i, 0)),
        out_shape=jax.ShapeDtypeStruct((N, H), jnp.float32),
    )(h, s, mx, w1, b1, w2, b2)


def _k_nodehead(x_ref, h_ref, nd1_ref, ndb1_ref, nd2_ref, ndb2_ref, wq_ref,
                nd_ref, q_ref):
    hn = _leaky(jnp.dot(h_ref[...], nd1_ref[...]) + ndb1_ref[...])
    nd_ref[...] = jnp.dot(hn, nd2_ref[...]) + ndb2_ref[...]
    nf1 = jnp.concatenate([x_ref[...], h_ref[...]], axis=1)
    q_ref[...] = jnp.dot(nf1, wq_ref[...])


def _nodehead(x, h, nd1, ndb1, nd2, ndb2, wq):
    return pl.pallas_call(
        _k_nodehead,
        grid=(N // NBLK,),
        in_specs=[
            pl.BlockSpec((NBLK, 128), lambda i: (i, 0)),
            pl.BlockSpec((NBLK, H), lambda i: (i, 0)),
            pl.BlockSpec((H, H), lambda i: (0, 0)),
            pl.BlockSpec((H,), lambda i: (0,)),
            pl.BlockSpec((H, 4), lambda i: (0, 0)),
            pl.BlockSpec((4,), lambda i: (0,)),
            pl.BlockSpec((128 + H, HEADS * DH), lambda i: (0, 0)),
        ],
        out_specs=[
            pl.BlockSpec((NBLK, 4), lambda i: (i, 0)),
            pl.BlockSpec((NBLK, HEADS * DH), lambda i: (i, 0)),
        ],
        out_shape=[
            jax.ShapeDtypeStruct((N, 4), jnp.float32),
            jax.ShapeDtypeStruct((N, HEADS * DH), jnp.float32),
        ],
    )(x, h, nd1, ndb1, nd2, ndb2, wq)


def _k_edgeattn(nfs_ref, nfd_ref, ce_ref, gq_ref, wk_ref, wv_ref, cd1_ref,
                cdb1_ref, cd2_ref, cdb2_ref, lg_ref, v_ref, cd_ref):
    sf = jnp.concatenate([nfs_ref[...], ce_ref[...]], axis=1)
    k = jnp.dot(sf, wk_ref[...])
    v = jnp.dot(sf, wv_ref[...])
    v_ref[...] = v
    q = gq_ref[...]
    qk = q * k
    blk = qk.shape[0]
    lg_ref[...] = jnp.sum(qk.reshape(blk, HEADS, DH), axis=-1) / np.sqrt(DH)
    ed = jnp.concatenate([nfs_ref[...], nfd_ref[...], ce_ref[...]], axis=1)
    hc = _leaky(jnp.dot(ed, cd1_ref[...]) + cdb1_ref[...])
    cd_ref[...] = jnp.dot(hc, cd2_ref[...]) + cdb2_ref[...]


def _edgeattn(nfs, nfd, ce, gq, wk, wv, cd1, cdb1, cd2, cdb2):
    D = 128 + H
    return pl.pallas_call(
        _k_edgeattn,
        grid=(E // EBLK,),
        in_specs=[
            pl.BlockSpec((EBLK, D), lambda i: (i, 0)),
            pl.BlockSpec((EBLK, D), lambda i: (i, 0)),
            pl.BlockSpec((EBLK, 7), lambda i: (i, 0)),
            pl.BlockSpec((EBLK, HEADS * DH), lambda i: (i, 0)),
            pl.BlockSpec((D + 7, HEADS * DH), lambda i: (0, 0)),
            pl.BlockSpec((D + 7, HEADS * DH), lambda i: (0, 0)),
            pl.BlockSpec((2 * D + 7, H), lambda i: (0, 0)),
            pl.BlockSpec((H,), lambda i: (0,)),
            pl.BlockSpec((H, 4), lambda i: (0, 0)),
            pl.BlockSpec((4,), lambda i: (0,)),
        ],
        out_specs=[
            pl.BlockSpec((EBLK, HEADS), lambda i: (i, 0)),
            pl.BlockSpec((EBLK, HEADS * DH), lambda i: (i, 0)),
            pl.BlockSpec((EBLK, 4), lambda i: (i, 0)),
        ],
        out_shape=[
            jax.ShapeDtypeStruct((E, HEADS), jnp.float32),
            jax.ShapeDtypeStruct((E, HEADS * DH), jnp.float32),
            jax.ShapeDtypeStruct((E, 4), jnp.float32),
        ],
    )(nfs, nfd, ce, gq, wk, wv, cd1, cdb1, cd2, cdb2)


def _k_aohead(x_ref, h_ref, agg_ref, w1_ref, b1_ref, w2_ref, b2_ref, o_ref):
    cat = jnp.concatenate([x_ref[...], h_ref[...], agg_ref[...]], axis=1)
    ha = _leaky(jnp.dot(cat, w1_ref[...]) + b1_ref[...])
    o_ref[...] = jnp.dot(ha, w2_ref[...]) + b2_ref[...]


def _aohead(x, h, agg, w1, b1, w2, b2):
    D = 128 + H
    return pl.pallas_call(
        _k_aohead,
        grid=(N // NBLK,),
        in_specs=[
            pl.BlockSpec((NBLK, 128), lambda i: (i, 0)),
            pl.BlockSpec((NBLK, H), lambda i: (i, 0)),
            pl.BlockSpec((NBLK, HEADS * DH), lambda i: (i, 0)),
            pl.BlockSpec((D + HEADS * DH, H), lambda i: (0, 0)),
            pl.BlockSpec((H,), lambda i: (0,)),
            pl.BlockSpec((H, 8), lambda i: (0, 0)),
            pl.BlockSpec((8,), lambda i: (0,)),
        ],
        out_specs=pl.BlockSpec((NBLK, 8), lambda i: (i, 0)),
        out_shape=jax.ShapeDtypeStruct((N, 8), jnp.float32),
    )(x, h, agg, w1, b1, w2, b2)


def kernel(x, edge_attr, cell_ef, params, edge_index):
    p = params
    src = edge_index[0]
    dst = edge_index[1]

    h = _dimup(x, p['dim_up_w'], p['dim_up_b'])
    for lp in p['nc']:
        gs = h[src]
        gd = h[dst]
        m = _msg(gs, gd, edge_attr, lp['msg_w1'], lp['msg_b1'],
                 lp['msg_w2'], lp['msg_b2'])
        gate = jax.nn.sigmoid(m[:, :1])
        f1 = m[:, 1:1 + H1] * gate
        f2 = m[:, 1 + H1:] * gate
        s = jax.ops.segment_sum(f1, dst, num_segments=N)
        mx = jax.ops.segment_max(f2, dst, num_segments=N)
        mx = jnp.where(jnp.isfinite(mx), mx, 0.0)
        h = _readout(h, s, mx, lp['ro_w1'], lp['ro_b1'],
                     lp['ro_w2'], lp['ro_b2'])

    net_delays, q_node = _nodehead(x, h, p['nd_w1'], p['nd_b1'],
                                   p['nd_w2'], p['nd_b2'], p['wq'])

    nf1 = jnp.concatenate([x, h], axis=1)
    nfs = nf1[src]
    nfd = nf1[dst]
    gq = q_node[dst]
    logits, v, cell_delays = _edgeattn(nfs, nfd, cell_ef, gq, p['wk'], p['wv'],
                                       p['cd_w1'], p['cd_b1'], p['cd_w2'],
                                       p['cd_b2'])
    lmax = jax.ops.segment_max(logits, dst, num_segments=N)
    lmax = jnp.where(jnp.isfinite(lmax), lmax, 0.0)
    ex = jnp.exp(logits - lmax[dst])
    den = jax.ops.segment_sum(ex, dst, num_segments=N) + 1e-9
    attn = ex / den[dst]
    agg = jax.ops.segment_sum(
        v.reshape(E, HEADS, DH) * attn[:, :, None], dst,
        num_segments=N).reshape(N, HEADS * DH)
    atslew = _aohead(x, h, agg, p['ao_w1'], p['ao_b1'], p['ao_w2'], p['ao_b2'])

    return net_delays, cell_delays, atslew


# speedup vs baseline: 1.1425x; 1.1322x over previous
import functools
import numpy as np
import jax
import jax.numpy as jnp
from jax import lax
from jax.experimental import pallas as pl
from jax.experimental.pallas import tpu as pltpu
from jax.experimental.pallas import tpu_sc as plsc

N = 10000
E = 320000
H = 64
H1 = 32
H2 = 32
HEADS = 8
DH = 8

NBLK = 2000
EBLK = 4000


def _leaky(v):
    return jax.nn.leaky_relu(v, 0.2)


_NW = 32


@functools.lru_cache(maxsize=None)
def _make_sc_gather(D, CH):
    per_w = E // _NW
    n_ch = per_w // CH
    mesh = plsc.VectorSubcoreMesh(core_axis_name="c", subcore_axis_name="s")

    @functools.partial(
        pl.kernel,
        mesh=mesh,
        out_type=jax.ShapeDtypeStruct((E, D), jnp.float32),
        scratch_types=[
            pltpu.VMEM((CH,), jnp.int32),
            pltpu.VMEM((CH, D), jnp.float32),
            pltpu.SemaphoreType.DMA,
        ],
    )
    def k(table_hbm, idx_hbm, out_hbm, idx_v, rows_v, sem):
        wid = lax.axis_index("s") * 2 + lax.axis_index("c")
        base = wid * per_w

        def body(i, carry):
            off = base + i * CH
            pltpu.sync_copy(idx_hbm.at[pl.ds(off, CH)], idx_v)
            pltpu.async_copy(table_hbm.at[idx_v], rows_v, sem).wait()
            pltpu.sync_copy(rows_v, out_hbm.at[pl.ds(off, CH)])
            return carry

        lax.fori_loop(0, n_ch, body, 0)

    return k


def _sc_gather(table, idx):
    D = table.shape[1]
    Dp = ((D + 127) // 128) * 128
    if Dp != D:
        table = jnp.pad(table, ((0, 0), (0, Dp - D)))
    return _make_sc_gather(Dp, 400)(table, idx)


def _k_dimup(x_ref, w_ref, b_ref, o_ref):
    o_ref[...] = _leaky(jnp.dot(x_ref[...], w_ref[...]) + b_ref[...])


def _dimup(x, w, b):
    return pl.pallas_call(
        _k_dimup,
        grid=(N // NBLK,),
        in_specs=[
            pl.BlockSpec((NBLK, 128), lambda i: (i, 0)),
            pl.BlockSpec((128, H), lambda i: (0, 0)),
            pl.BlockSpec((H,), lambda i: (0,)),
        ],
        out_specs=pl.BlockSpec((NBLK, H), lambda i: (i, 0)),
        out_shape=jax.ShapeDtypeStruct((N, H), jnp.float32),
    )(x, w, b)


def _k_msg(gs_ref, gd_ref, ea_ref, w1_ref, b1_ref, w2_ref, b2_ref, m_ref):
    m_in = jnp.concatenate([gs_ref[:, :H], gd_ref[:, :H], ea_ref[...]], axis=1)
    hm = _leaky(jnp.dot(m_in, w1_ref[...]) + b1_ref[...])
    m_ref[...] = jnp.dot(hm, w2_ref[...]) + b2_ref[...]


def _msg(gs, gd, ea, w1, b1, w2, b2):
    return pl.pallas_call(
        _k_msg,
        grid=(E // EBLK,),
        in_specs=[
            pl.BlockSpec((EBLK, 128), lambda i: (i, 0)),
            pl.BlockSpec((EBLK, 128), lambda i: (i, 0)),
            pl.BlockSpec((EBLK, 2), lambda i: (i, 0)),
            pl.BlockSpec((2 * H + 2, H), lambda i: (0, 0)),
            pl.BlockSpec((H,), lambda i: (0,)),
            pl.BlockSpec((H, 1 + H1 + H2), lambda i: (0, 0)),
            pl.BlockSpec((1 + H1 + H2,), lambda i: (0,)),
        ],
        out_specs=pl.BlockSpec((EBLK, 1 + H1 + H2), lambda i: (i, 0)),
        out_shape=jax.ShapeDtypeStruct((E, 1 + H1 + H2), jnp.float32),
    )(gs, gd, ea, w1, b1, w2, b2)


def _k_readout(h_ref, s_ref, mx_ref, w1_ref, b1_ref, w2_ref, b2_ref, o_ref):
    r = jnp.concatenate([h_ref[...], s_ref[...], mx_ref[...]], axis=1)
    hr = _leaky(jnp.dot(r, w1_ref[...]) + b1_ref[...])
    o_ref[...] = jnp.dot(hr, w2_ref[...]) + b2_ref[...] + h_ref[...]


def _readout(h, s, mx, w1, b1, w2, b2):
    return pl.pallas_call(
        _k_readout,
        grid=(N // NBLK,),
        in_specs=[
            pl.BlockSpec((NBLK, H), lambda i: (i, 0)),
            pl.BlockSpec((NBLK, H1), lambda i: (i, 0)),
            pl.BlockSpec((NBLK, H2), lambda i: (i, 0)),
            pl.BlockSpec((H + H1 + H2, H), lambda i: (0, 0)),
            pl.BlockSpec((H,), lambda i: (0,)),
            pl.BlockSpec((H, H), lambda i: (0, 0)),
            pl.BlockSpec((H,), lambda i: (0,)),
        ],
        out_specs=pl.BlockSpec((NBLK, H), lambda i: (i, 0)),
        out_shape=jax.ShapeDtypeStruct((N, H), jnp.float32),
    )(h, s, mx, w1, b1, w2, b2)


def _k_nodehead(x_ref, h_ref, nd1_ref, ndb1_ref, nd2_ref, ndb2_ref, wq_ref,
                nd_ref, q_ref):
    hn = _leaky(jnp.dot(h_ref[...], nd1_ref[...]) + ndb1_ref[...])
    nd_ref[...] = jnp.dot(hn, nd2_ref[...]) + ndb2_ref[...]
    nf1 = jnp.concatenate([x_ref[...], h_ref[...]], axis=1)
    q_ref[...] = jnp.dot(nf1, wq_ref[...])


def _nodehead(x, h, nd1, ndb1, nd2, ndb2, wq):
    return pl.pallas_call(
        _k_nodehead,
        grid=(N // NBLK,),
        in_specs=[
            pl.BlockSpec((NBLK, 128), lambda i: (i, 0)),
            pl.BlockSpec((NBLK, H), lambda i: (i, 0)),
            pl.BlockSpec((H, H), lambda i: (0, 0)),
            pl.BlockSpec((H,), lambda i: (0,)),
            pl.BlockSpec((H, 4), lambda i: (0, 0)),
            pl.BlockSpec((4,), lambda i: (0,)),
            pl.BlockSpec((128 + H, HEADS * DH), lambda i: (0, 0)),
        ],
        out_specs=[
            pl.BlockSpec((NBLK, 4), lambda i: (i, 0)),
            pl.BlockSpec((NBLK, HEADS * DH), lambda i: (i, 0)),
        ],
        out_shape=[
            jax.ShapeDtypeStruct((N, 4), jnp.float32),
            jax.ShapeDtypeStruct((N, HEADS * DH), jnp.float32),
        ],
    )(x, h, nd1, ndb1, nd2, ndb2, wq)


def _k_edgeattn(nfs_ref, nfd_ref, ce_ref, gq_ref, wk_ref, wv_ref, cd1_ref,
                cdb1_ref, cd2_ref, cdb2_ref, lg_ref, v_ref, cd_ref):
    D = 128 + H
    sf = jnp.concatenate([nfs_ref[:, :D], ce_ref[...]], axis=1)
    k = jnp.dot(sf, wk_ref[...])
    v = jnp.dot(sf, wv_ref[...])
    v_ref[...] = v
    q = gq_ref[:, :HEADS * DH]
    qk = q * k
    blk = qk.shape[0]
    lg_ref[...] = jnp.sum(qk.reshape(blk, HEADS, DH), axis=-1) / np.sqrt(DH)
    ed = jnp.concatenate([nfs_ref[:, :D], nfd_ref[:, :D], ce_ref[...]], axis=1)
    hc = _leaky(jnp.dot(ed, cd1_ref[...]) + cdb1_ref[...])
    cd_ref[...] = jnp.dot(hc, cd2_ref[...]) + cdb2_ref[...]


def _edgeattn(nfs, nfd, ce, gq, wk, wv, cd1, cdb1, cd2, cdb2):
    D = 128 + H
    return pl.pallas_call(
        _k_edgeattn,
        grid=(E // EBLK,),
        in_specs=[
            pl.BlockSpec((EBLK, 256), lambda i: (i, 0)),
            pl.BlockSpec((EBLK, 256), lambda i: (i, 0)),
            pl.BlockSpec((EBLK, 7), lambda i: (i, 0)),
            pl.BlockSpec((EBLK, 128), lambda i: (i, 0)),
            pl.BlockSpec((D + 7, HEADS * DH), lambda i: (0, 0)),
            pl.BlockSpec((D + 7, HEADS * DH), lambda i: (0, 0)),
            pl.BlockSpec((2 * D + 7, H), lambda i: (0, 0)),
            pl.BlockSpec((H,), lambda i: (0,)),
            pl.BlockSpec((H, 4), lambda i: (0, 0)),
            pl.BlockSpec((4,), lambda i: (0,)),
        ],
        out_specs=[
            pl.BlockSpec((EBLK, HEADS), lambda i: (i, 0)),
            pl.BlockSpec((EBLK, HEADS * DH), lambda i: (i, 0)),
            pl.BlockSpec((EBLK, 4), lambda i: (i, 0)),
        ],
        out_shape=[
            jax.ShapeDtypeStruct((E, HEADS), jnp.float32),
            jax.ShapeDtypeStruct((E, HEADS * DH), jnp.float32),
            jax.ShapeDtypeStruct((E, 4), jnp.float32),
        ],
    )(nfs, nfd, ce, gq, wk, wv, cd1, cdb1, cd2, cdb2)


def _k_aohead(x_ref, h_ref, agg_ref, w1_ref, b1_ref, w2_ref, b2_ref, o_ref):
    cat = jnp.concatenate([x_ref[...], h_ref[...], agg_ref[...]], axis=1)
    ha = _leaky(jnp.dot(cat, w1_ref[...]) + b1_ref[...])
    o_ref[...] = jnp.dot(ha, w2_ref[...]) + b2_ref[...]


def _aohead(x, h, agg, w1, b1, w2, b2):
    D = 128 + H
    return pl.pallas_call(
        _k_aohead,
        grid=(N // NBLK,),
        in_specs=[
            pl.BlockSpec((NBLK, 128), lambda i: (i, 0)),
            pl.BlockSpec((NBLK, H), lambda i: (i, 0)),
            pl.BlockSpec((NBLK, HEADS * DH), lambda i: (i, 0)),
            pl.BlockSpec((D + HEADS * DH, H), lambda i: (0, 0)),
            pl.BlockSpec((H,), lambda i: (0,)),
            pl.BlockSpec((H, 8), lambda i: (0, 0)),
            pl.BlockSpec((8,), lambda i: (0,)),
        ],
        out_specs=pl.BlockSpec((NBLK, 8), lambda i: (i, 0)),
        out_shape=jax.ShapeDtypeStruct((N, 8), jnp.float32),
    )(x, h, agg, w1, b1, w2, b2)


def kernel(x, edge_attr, cell_ef, params, edge_index):
    p = params
    src = edge_index[0]
    dst = edge_index[1]

    h = _dimup(x, p['dim_up_w'], p['dim_up_b'])
    for lp in p['nc']:
        gs = _sc_gather(h, src)
        gd = _sc_gather(h, dst)
        m = _msg(gs, gd, edge_attr, lp['msg_w1'], lp['msg_b1'],
                 lp['msg_w2'], lp['msg_b2'])
        gate = jax.nn.sigmoid(m[:, :1])
        f1 = m[:, 1:1 + H1] * gate
        f2 = m[:, 1 + H1:] * gate
        s = jax.ops.segment_sum(f1, dst, num_segments=N)
        mx = jax.ops.segment_max(f2, dst, num_segments=N)
        mx = jnp.where(jnp.isfinite(mx), mx, 0.0)
        h = _readout(h, s, mx, lp['ro_w1'], lp['ro_b1'],
                     lp['ro_w2'], lp['ro_b2'])

    net_delays, q_node = _nodehead(x, h, p['nd_w1'], p['nd_b1'],
                                   p['nd_w2'], p['nd_b2'], p['wq'])

    nf1 = jnp.concatenate([x, h], axis=1)
    nfs = _sc_gather(nf1, src)
    nfd = _sc_gather(nf1, dst)
    gq = _sc_gather(q_node, dst)
    logits, v, cell_delays = _edgeattn(nfs, nfd, cell_ef, gq, p['wk'], p['wv'],
                                       p['cd_w1'], p['cd_b1'], p['cd_w2'],
                                       p['cd_b2'])
    lmax = jax.ops.segment_max(logits, dst, num_segments=N)
    lmax = jnp.where(jnp.isfinite(lmax), lmax, 0.0)
    ex = jnp.exp(logits - _sc_gather(lmax, dst)[:, :HEADS])
    den = jax.ops.segment_sum(ex, dst, num_segments=N) + 1e-9
    attn = ex / _sc_gather(den, dst)[:, :HEADS]
    agg = jax.ops.segment_sum(
        v.reshape(E, HEADS, DH) * attn[:, :, None], dst,
        num_segments=N).reshape(N, HEADS * DH)
    atslew = _aohead(x, h, agg, p['ao_w1'], p['ao_b1'], p['ao_w2'], p['ao_b2'])

    return net_delays, cell_delays, atslew


# speedup vs baseline: 1.1533x; 1.0095x over previous
import functools
import numpy as np
import jax
import jax.numpy as jnp
from jax import lax
from jax.experimental import pallas as pl
from jax.experimental.pallas import tpu as pltpu
from jax.experimental.pallas import tpu_sc as plsc

N = 10000
E = 320000
H = 64
H1 = 32
H2 = 32
HEADS = 8
DH = 8

NBLK = 2000
EBLK = 4000


def _leaky(v):
    return jax.nn.leaky_relu(v, 0.2)


_NW = 32


@functools.lru_cache(maxsize=None)
def _make_sc_gather(D, CH):
    per_w = E // _NW
    n_ch = per_w // CH
    mesh = plsc.VectorSubcoreMesh(core_axis_name="c", subcore_axis_name="s")

    @functools.partial(
        pl.kernel,
        mesh=mesh,
        out_type=jax.ShapeDtypeStruct((E, D), jnp.float32),
        scratch_types=[
            pltpu.VMEM((CH,), jnp.int32),
            pltpu.VMEM((CH, D), jnp.float32),
            pltpu.SemaphoreType.DMA,
        ],
    )
    def k(table_hbm, idx_hbm, out_hbm, idx_v, rows_v, sem):
        wid = lax.axis_index("s") * 2 + lax.axis_index("c")
        base = wid * per_w

        def body(i, carry):
            off = base + i * CH
            pltpu.sync_copy(idx_hbm.at[pl.ds(off, CH)], idx_v)
            pltpu.async_copy(table_hbm.at[idx_v], rows_v, sem).wait()
            pltpu.sync_copy(rows_v, out_hbm.at[pl.ds(off, CH)])
            return carry

        lax.fori_loop(0, n_ch, body, 0)

    return k


def _sc_gather(table, idx):
    D = table.shape[1]
    Dp = ((D + 127) // 128) * 128
    if Dp != D:
        table = jnp.pad(table, ((0, 0), (0, Dp - D)))
    return _make_sc_gather(Dp, 400)(table, idx)


@functools.lru_cache(maxsize=None)
def _make_sc_segmax(D, R, CH):
    per_r = E // R
    n_ch = per_r // CH
    n_vec = CH // 16
    rows = N * 8 // 128
    mesh = plsc.VectorSubcoreMesh(core_axis_name="c", subcore_axis_name="s")

    @functools.partial(
        pl.kernel,
        mesh=mesh,
        compiler_params=pltpu.CompilerParams(needs_layout_passes=False),
        out_type=jax.ShapeDtypeStruct((R * D, rows, 128), jnp.float32),
        scratch_types=[
            pltpu.VMEM((rows, 128), jnp.float32),
            pltpu.VMEM((CH,), jnp.int32),
            pltpu.VMEM((CH,), jnp.float32),
        ],
    )
    def k(vals_hbm, idx_hbm, ninf_hbm, out_hbm, acc_v, idx_v, val_v):
        wid = lax.axis_index("s") * 2 + lax.axis_index("c")
        d = wid % D
        r = wid // D
        lane = lax.iota(jnp.int32, 16)
        colv = lax.bitwise_and(lane, 7)
        m_lo = lane < 8
        m_hi = lane >= 8
        pltpu.sync_copy(ninf_hbm, acc_v)

        def chunk_body(c, carry):
            off = r * per_r + c * CH
            pltpu.sync_copy(idx_hbm.at[pl.ds(off, CH)], idx_v)
            pltpu.sync_copy(vals_hbm.at[pl.ds(d * E + off, CH)], val_v)

            def vec_body(j, carry2):
                idxv = idx_v[pl.ds(j * 16, 16)]
                valv = val_v[pl.ds(j * 16, 16)]
                flat = lax.shift_left(idxv, 3) + colv
                rowv = lax.shift_right_logical(flat, 7)
                lanev = lax.bitwise_and(flat, 127)
                g = plsc.load_gather(acc_v, [rowv, lanev])
                plsc.store_scatter(acc_v, [rowv, lanev],
                                   jnp.maximum(g, valv), mask=m_lo)
                g2 = plsc.load_gather(acc_v, [rowv, lanev])
                plsc.store_scatter(acc_v, [rowv, lanev],
                                   jnp.maximum(g2, valv), mask=m_hi)
                return carry2

            lax.fori_loop(0, n_vec, vec_body, 0)
            return carry

        lax.fori_loop(0, n_ch, chunk_body, 0)
        pltpu.sync_copy(acc_v, out_hbm.at[r * D + d])

    return k


_NEG_INF_ACC = None


def _sc_segmax(vals, dst):
    D = vals.shape[1]
    R = _NW // D
    vals_flat = vals.T.reshape(-1)
    ninf = jnp.full((N * 8 // 128, 128), -jnp.inf, jnp.float32)
    out = _make_sc_segmax(D, R, 4000)(vals_flat, dst, ninf)
    part = out.reshape(R, D, N, 8)
    return jnp.max(part, axis=(0, 3)).T


def _k_dimup(x_ref, w_ref, b_ref, o_ref):
    o_ref[...] = _leaky(jnp.dot(x_ref[...], w_ref[...]) + b_ref[...])


def _dimup(x, w, b):
    return pl.pallas_call(
        _k_dimup,
        grid=(N // NBLK,),
        in_specs=[
            pl.BlockSpec((NBLK, 128), lambda i: (i, 0)),
            pl.BlockSpec((128, H), lambda i: (0, 0)),
            pl.BlockSpec((H,), lambda i: (0,)),
        ],
        out_specs=pl.BlockSpec((NBLK, H), lambda i: (i, 0)),
        out_shape=jax.ShapeDtypeStruct((N, H), jnp.float32),
    )(x, w, b)


def _k_msg(gs_ref, gd_ref, ea_ref, w1_ref, b1_ref, w2_ref, b2_ref, m_ref):
    m_in = jnp.concatenate([gs_ref[:, :H], gd_ref[:, :H], ea_ref[...]], axis=1)
    hm = _leaky(jnp.dot(m_in, w1_ref[...]) + b1_ref[...])
    m_ref[...] = jnp.dot(hm, w2_ref[...]) + b2_ref[...]


def _msg(gs, gd, ea, w1, b1, w2, b2):
    return pl.pallas_call(
        _k_msg,
        grid=(E // EBLK,),
        in_specs=[
            pl.BlockSpec((EBLK, 128), lambda i: (i, 0)),
            pl.BlockSpec((EBLK, 128), lambda i: (i, 0)),
            pl.BlockSpec((EBLK, 2), lambda i: (i, 0)),
            pl.BlockSpec((2 * H + 2, H), lambda i: (0, 0)),
            pl.BlockSpec((H,), lambda i: (0,)),
            pl.BlockSpec((H, 1 + H1 + H2), lambda i: (0, 0)),
            pl.BlockSpec((1 + H1 + H2,), lambda i: (0,)),
        ],
        out_specs=pl.BlockSpec((EBLK, 1 + H1 + H2), lambda i: (i, 0)),
        out_shape=jax.ShapeDtypeStruct((E, 1 + H1 + H2), jnp.float32),
    )(gs, gd, ea, w1, b1, w2, b2)


def _k_readout(h_ref, s_ref, mx_ref, w1_ref, b1_ref, w2_ref, b2_ref, o_ref):
    r = jnp.concatenate([h_ref[...], s_ref[...], mx_ref[...]], axis=1)
    hr = _leaky(jnp.dot(r, w1_ref[...]) + b1_ref[...])
    o_ref[...] = jnp.dot(hr, w2_ref[...]) + b2_ref[...] + h_ref[...]


def _readout(h, s, mx, w1, b1, w2, b2):
    return pl.pallas_call(
        _k_readout,
        grid=(N // NBLK,),
        in_specs=[
            pl.BlockSpec((NBLK, H), lambda i: (i, 0)),
            pl.BlockSpec((NBLK, H1), lambda i: (i, 0)),
            pl.BlockSpec((NBLK, H2), lambda i: (i, 0)),
            pl.BlockSpec((H + H1 + H2, H), lambda i: (0, 0)),
            pl.BlockSpec((H,), lambda i: (0,)),
            pl.BlockSpec((H, H), lambda i: (0, 0)),
            pl.BlockSpec((H,), lambda i: (0,)),
        ],
        out_specs=pl.BlockSpec((NBLK, H), lambda i: (i, 0)),
        out_shape=jax.ShapeDtypeStruct((N, H), jnp.float32),
    )(h, s, mx, w1, b1, w2, b2)


def _k_nodehead(x_ref, h_ref, nd1_ref, ndb1_ref, nd2_ref, ndb2_ref, wq_ref,
                nd_ref, q_ref):
    hn = _leaky(jnp.dot(h_ref[...], nd1_ref[...]) + ndb1_ref[...])
    nd_ref[...] = jnp.dot(hn, nd2_ref[...]) + ndb2_ref[...]
    nf1 = jnp.concatenate([x_ref[...], h_ref[...]], axis=1)
    q_ref[...] = jnp.dot(nf1, wq_ref[...])


def _nodehead(x, h, nd1, ndb1, nd2, ndb2, wq):
    return pl.pallas_call(
        _k_nodehead,
        grid=(N // NBLK,),
        in_specs=[
            pl.BlockSpec((NBLK, 128), lambda i: (i, 0)),
            pl.BlockSpec((NBLK, H), lambda i: (i, 0)),
            pl.BlockSpec((H, H), lambda i: (0, 0)),
            pl.BlockSpec((H,), lambda i: (0,)),
            pl.BlockSpec((H, 4), lambda i: (0, 0)),
            pl.BlockSpec((4,), lambda i: (0,)),
            pl.BlockSpec((128 + H, HEADS * DH), lambda i: (0, 0)),
        ],
        out_specs=[
            pl.BlockSpec((NBLK, 4), lambda i: (i, 0)),
            pl.BlockSpec((NBLK, HEADS * DH), lambda i: (i, 0)),
        ],
        out_shape=[
            jax.ShapeDtypeStruct((N, 4), jnp.float32),
            jax.ShapeDtypeStruct((N, HEADS * DH), jnp.float32),
        ],
    )(x, h, nd1, ndb1, nd2, ndb2, wq)


def _k_edgeattn(nfs_ref, nfd_ref, ce_ref, gq_ref, wk_ref, wv_ref, cd1_ref,
                cdb1_ref, cd2_ref, cdb2_ref, lg_ref, v_ref, cd_ref):
    D = 128 + H
    sf = jnp.concatenate([nfs_ref[:, :D], ce_ref[...]], axis=1)
    k = jnp.dot(sf, wk_ref[...])
    v = jnp.dot(sf, wv_ref[...])
    v_ref[...] = v
    q = gq_ref[:, :HEADS * DH]
    qk = q * k
    blk = qk.shape[0]
    lg_ref[...] = jnp.sum(qk.reshape(blk, HEADS, DH), axis=-1) / np.sqrt(DH)
    ed = jnp.concatenate([nfs_ref[:, :D], nfd_ref[:, :D], ce_ref[...]], axis=1)
    hc = _leaky(jnp.dot(ed, cd1_ref[...]) + cdb1_ref[...])
    cd_ref[...] = jnp.dot(hc, cd2_ref[...]) + cdb2_ref[...]


def _edgeattn(nfs, nfd, ce, gq, wk, wv, cd1, cdb1, cd2, cdb2):
    D = 128 + H
    return pl.pallas_call(
        _k_edgeattn,
        grid=(E // EBLK,),
        in_specs=[
            pl.BlockSpec((EBLK, 256), lambda i: (i, 0)),
            pl.BlockSpec((EBLK, 256), lambda i: (i, 0)),
            pl.BlockSpec((EBLK, 7), lambda i: (i, 0)),
            pl.BlockSpec((EBLK, 128), lambda i: (i, 0)),
            pl.BlockSpec((D + 7, HEADS * DH), lambda i: (0, 0)),
            pl.BlockSpec((D + 7, HEADS * DH), lambda i: (0, 0)),
            pl.BlockSpec((2 * D + 7, H), lambda i: (0, 0)),
            pl.BlockSpec((H,), lambda i: (0,)),
            pl.BlockSpec((H, 4), lambda i: (0, 0)),
            pl.BlockSpec((4,), lambda i: (0,)),
        ],
        out_specs=[
            pl.BlockSpec((EBLK, HEADS), lambda i: (i, 0)),
            pl.BlockSpec((EBLK, HEADS * DH), lambda i: (i, 0)),
            pl.BlockSpec((EBLK, 4), lambda i: (i, 0)),
        ],
        out_shape=[
            jax.ShapeDtypeStruct((E, HEADS), jnp.float32),
            jax.ShapeDtypeStruct((E, HEADS * DH), jnp.float32),
            jax.ShapeDtypeStruct((E, 4), jnp.float32),
        ],
    )(nfs, nfd, ce, gq, wk, wv, cd1, cdb1, cd2, cdb2)


def _k_aohead(x_ref, h_ref, agg_ref, w1_ref, b1_ref, w2_ref, b2_ref, o_ref):
    cat = jnp.concatenate([x_ref[...], h_ref[...], agg_ref[...]], axis=1)
    ha = _leaky(jnp.dot(cat, w1_ref[...]) + b1_ref[...])
    o_ref[...] = jnp.dot(ha, w2_ref[...]) + b2_ref[...]


def _aohead(x, h, agg, w1, b1, w2, b2):
    D = 128 + H
    return pl.pallas_call(
        _k_aohead,
        grid=(N // NBLK,),
        in_specs=[
            pl.BlockSpec((NBLK, 128), lambda i: (i, 0)),
            pl.BlockSpec((NBLK, H), lambda i: (i, 0)),
            pl.BlockSpec((NBLK, HEADS * DH), lambda i: (i, 0)),
            pl.BlockSpec((D + HEADS * DH, H), lambda i: (0, 0)),
            pl.BlockSpec((H,), lambda i: (0,)),
            pl.BlockSpec((H, 8), lambda i: (0, 0)),
            pl.BlockSpec((8,), lambda i: (0,)),
        ],
        out_specs=pl.BlockSpec((NBLK, 8), lambda i: (i, 0)),
        out_shape=jax.ShapeDtypeStruct((N, 8), jnp.float32),
    )(x, h, agg, w1, b1, w2, b2)


def kernel(x, edge_attr, cell_ef, params, edge_index):
    p = params
    src = edge_index[0]
    dst = edge_index[1]

    h = _dimup(x, p['dim_up_w'], p['dim_up_b'])
    for lp in p['nc']:
        gs = _sc_gather(h, src)
        gd = _sc_gather(h, dst)
        m = _msg(gs, gd, edge_attr, lp['msg_w1'], lp['msg_b1'],
                 lp['msg_w2'], lp['msg_b2'])
        gate = jax.nn.sigmoid(m[:, :1])
        f1 = m[:, 1:1 + H1] * gate
        f2 = m[:, 1 + H1:] * gate
        s = jax.ops.segment_sum(f1, dst, num_segments=N)
        mx = _sc_segmax(f2, dst)
        mx = jnp.where(jnp.isfinite(mx), mx, 0.0)
        h = _readout(h, s, mx, lp['ro_w1'], lp['ro_b1'],
                     lp['ro_w2'], lp['ro_b2'])

    net_delays, q_node = _nodehead(x, h, p['nd_w1'], p['nd_b1'],
                                   p['nd_w2'], p['nd_b2'], p['wq'])

    nf1 = jnp.concatenate([x, h], axis=1)
    nfs = _sc_gather(nf1, src)
    nfd = _sc_gather(nf1, dst)
    gq = _sc_gather(q_node, dst)
    logits, v, cell_delays = _edgeattn(nfs, nfd, cell_ef, gq, p['wk'], p['wv'],
                                       p['cd_w1'], p['cd_b1'], p['cd_w2'],
                                       p['cd_b2'])
    lmax = _sc_segmax(logits, dst)
    lmax = jnp.where(jnp.isfinite(lmax), lmax, 0.0)
    ex = jnp.exp(logits - _sc_gather(lmax, dst)[:, :HEADS])
    den = jax.ops.segment_sum(ex, dst, num_segments=N) + 1e-9
    attn = ex / _sc_gather(den, dst)[:, :HEADS]
    agg = jax.ops.segment_sum(
        v.reshape(E, HEADS, DH) * attn[:, :, None], dst,
        num_segments=N).reshape(N, HEADS * DH)
    atslew = _aohead(x, h, agg, p['ao_w1'], p['ao_b1'], p['ao_w2'], p['ao_b2'])

    return net_delays, cell_delays, atslew


# speedup vs baseline: 4.3169x; 3.7431x over previous
import functools
import numpy as np
import jax
import jax.numpy as jnp
from jax import lax
from jax.experimental import pallas as pl
from jax.experimental.pallas import tpu as pltpu
from jax.experimental.pallas import tpu_sc as plsc

N = 10000
E = 320000
H = 64
H1 = 32
H2 = 32
HEADS = 8
DH = 8

NBLK = 2000
EBLK = 4000


def _leaky(v):
    return jax.nn.leaky_relu(v, 0.2)


_NW = 32


@functools.lru_cache(maxsize=None)
def _make_sc_gather(D, CH):
    per_w = E // _NW
    n_ch = per_w // CH
    mesh = plsc.VectorSubcoreMesh(core_axis_name="c", subcore_axis_name="s")

    @functools.partial(
        pl.kernel,
        mesh=mesh,
        out_type=jax.ShapeDtypeStruct((E, D), jnp.float32),
        scratch_types=[
            pltpu.VMEM((CH,), jnp.int32),
            pltpu.VMEM((CH, D), jnp.float32),
            pltpu.SemaphoreType.DMA,
        ],
    )
    def k(table_hbm, idx_hbm, out_hbm, idx_v, rows_v, sem):
        wid = lax.axis_index("s") * 2 + lax.axis_index("c")
        base = wid * per_w

        def body(i, carry):
            off = base + i * CH
            pltpu.sync_copy(idx_hbm.at[pl.ds(off, CH)], idx_v)
            pltpu.async_copy(table_hbm.at[idx_v], rows_v, sem).wait()
            pltpu.sync_copy(rows_v, out_hbm.at[pl.ds(off, CH)])
            return carry

        lax.fori_loop(0, n_ch, body, 0)

    return k


def _sc_gather(table, idx):
    D = table.shape[1]
    Dp = ((D + 127) // 128) * 128
    if Dp != D:
        table = jnp.pad(table, ((0, 0), (0, Dp - D)))
    return _make_sc_gather(Dp, 400)(table, idx)


@functools.lru_cache(maxsize=None)
def _make_sc_segmax(D, R, CH):
    per_r = E // R
    n_ch = per_r // CH
    n_vec = CH // 16
    rows = N * 8 // 128
    mesh = plsc.VectorSubcoreMesh(core_axis_name="c", subcore_axis_name="s")

    @functools.partial(
        pl.kernel,
        mesh=mesh,
        compiler_params=pltpu.CompilerParams(needs_layout_passes=False),
        out_type=jax.ShapeDtypeStruct((R * D, rows, 128), jnp.float32),
        scratch_types=[
            pltpu.VMEM((rows, 128), jnp.float32),
            pltpu.VMEM((CH,), jnp.int32),
            pltpu.VMEM((CH,), jnp.float32),
        ],
    )
    def k(vals_hbm, idx_hbm, ninf_hbm, out_hbm, acc_v, idx_v, val_v):
        wid = lax.axis_index("s") * 2 + lax.axis_index("c")
        d = wid % D
        r = wid // D
        lane = lax.iota(jnp.int32, 16)
        colv = lax.bitwise_and(lane, 7)
        m_lo = lane < 8
        m_hi = lane >= 8
        pltpu.sync_copy(ninf_hbm, acc_v)

        def chunk_body(c, carry):
            off = r * per_r + c * CH
            pltpu.sync_copy(idx_hbm.at[pl.ds(off, CH)], idx_v)
            pltpu.sync_copy(vals_hbm.at[pl.ds(d * E + off, CH)], val_v)

            def vec_body(j, carry2):
                idxv = idx_v[pl.ds(j * 16, 16)]
                valv = val_v[pl.ds(j * 16, 16)]
                flat = lax.shift_left(idxv, 3) + colv
                rowv = lax.shift_right_logical(flat, 7)
                lanev = lax.bitwise_and(flat, 127)
                g = plsc.load_gather(acc_v, [rowv, lanev])
                plsc.store_scatter(acc_v, [rowv, lanev],
                                   jnp.maximum(g, valv), mask=m_lo)
                g2 = plsc.load_gather(acc_v, [rowv, lanev])
                plsc.store_scatter(acc_v, [rowv, lanev],
                                   jnp.maximum(g2, valv), mask=m_hi)
                return carry2

            lax.fori_loop(0, n_vec, vec_body, 0)
            return carry

        lax.fori_loop(0, n_ch, chunk_body, 0)
        pltpu.sync_copy(acc_v, out_hbm.at[r * D + d])

    return k


_NEG_INF_ACC = None


def _sc_segmax(vals, dst):
    D = vals.shape[1]
    R = _NW // D
    vals_flat = vals.T.reshape(-1)
    ninf = jnp.full((N * 8 // 128, 128), -jnp.inf, jnp.float32)
    out = _make_sc_segmax(D, R, 4000)(vals_flat, dst, ninf)
    part = out.reshape(R, D, N, 8)
    return jnp.max(part, axis=(0, 3)).T


def _k_dimup(x_ref, w_ref, b_ref, o_ref):
    o_ref[...] = _leaky(jnp.dot(x_ref[...], w_ref[...]) + b_ref[...])


def _dimup(x, w, b):
    return pl.pallas_call(
        _k_dimup,
        grid=(N // NBLK,),
        in_specs=[
            pl.BlockSpec((NBLK, 128), lambda i: (i, 0)),
            pl.BlockSpec((128, H), lambda i: (0, 0)),
            pl.BlockSpec((H,), lambda i: (0,)),
        ],
        out_specs=pl.BlockSpec((NBLK, H), lambda i: (i, 0)),
        out_shape=jax.ShapeDtypeStruct((N, H), jnp.float32),
    )(x, w, b)


def _k_msg(gs_ref, gd_ref, ea_ref, w1_ref, b1_ref, w2_ref, b2_ref, m_ref):
    m_in = jnp.concatenate([gs_ref[:, :H], gd_ref[:, :H], ea_ref[...]], axis=1)
    hm = _leaky(jnp.dot(m_in, w1_ref[...]) + b1_ref[...])
    m_ref[...] = jnp.dot(hm, w2_ref[...]) + b2_ref[...]


def _msg(gs, gd, ea, w1, b1, w2, b2):
    return pl.pallas_call(
        _k_msg,
        grid=(E // EBLK,),
        in_specs=[
            pl.BlockSpec((EBLK, 128), lambda i: (i, 0)),
            pl.BlockSpec((EBLK, 128), lambda i: (i, 0)),
            pl.BlockSpec((EBLK, 2), lambda i: (i, 0)),
            pl.BlockSpec((2 * H + 2, H), lambda i: (0, 0)),
            pl.BlockSpec((H,), lambda i: (0,)),
            pl.BlockSpec((H, 1 + H1 + H2), lambda i: (0, 0)),
            pl.BlockSpec((1 + H1 + H2,), lambda i: (0,)),
        ],
        out_specs=pl.BlockSpec((EBLK, 1 + H1 + H2), lambda i: (i, 0)),
        out_shape=jax.ShapeDtypeStruct((E, 1 + H1 + H2), jnp.float32),
    )(gs, gd, ea, w1, b1, w2, b2)


def _k_readout(h_ref, s_ref, mx_ref, w1_ref, b1_ref, w2_ref, b2_ref, o_ref):
    r = jnp.concatenate([h_ref[...], s_ref[...], mx_ref[...]], axis=1)
    hr = _leaky(jnp.dot(r, w1_ref[...]) + b1_ref[...])
    o_ref[...] = jnp.dot(hr, w2_ref[...]) + b2_ref[...] + h_ref[...]


def _readout(h, s, mx, w1, b1, w2, b2):
    return pl.pallas_call(
        _k_readout,
        grid=(N // NBLK,),
        in_specs=[
            pl.BlockSpec((NBLK, H), lambda i: (i, 0)),
            pl.BlockSpec((NBLK, H1), lambda i: (i, 0)),
            pl.BlockSpec((NBLK, H2), lambda i: (i, 0)),
            pl.BlockSpec((H + H1 + H2, H), lambda i: (0, 0)),
            pl.BlockSpec((H,), lambda i: (0,)),
            pl.BlockSpec((H, H), lambda i: (0, 0)),
            pl.BlockSpec((H,), lambda i: (0,)),
        ],
        out_specs=pl.BlockSpec((NBLK, H), lambda i: (i, 0)),
        out_shape=jax.ShapeDtypeStruct((N, H), jnp.float32),
    )(h, s, mx, w1, b1, w2, b2)


def _k_nodehead(x_ref, h_ref, nd1_ref, ndb1_ref, nd2_ref, ndb2_ref, wq_ref,
                nd_ref, q_ref):
    hn = _leaky(jnp.dot(h_ref[...], nd1_ref[...]) + ndb1_ref[...])
    nd_ref[...] = jnp.dot(hn, nd2_ref[...]) + ndb2_ref[...]
    nf1 = jnp.concatenate([x_ref[...], h_ref[...]], axis=1)
    q_ref[...] = jnp.dot(nf1, wq_ref[...])


def _nodehead(x, h, nd1, ndb1, nd2, ndb2, wq):
    return pl.pallas_call(
        _k_nodehead,
        grid=(N // NBLK,),
        in_specs=[
            pl.BlockSpec((NBLK, 128), lambda i: (i, 0)),
            pl.BlockSpec((NBLK, H), lambda i: (i, 0)),
            pl.BlockSpec((H, H), lambda i: (0, 0)),
            pl.BlockSpec((H,), lambda i: (0,)),
            pl.BlockSpec((H, 4), lambda i: (0, 0)),
            pl.BlockSpec((4,), lambda i: (0,)),
            pl.BlockSpec((128 + H, HEADS * DH), lambda i: (0, 0)),
        ],
        out_specs=[
            pl.BlockSpec((NBLK, 4), lambda i: (i, 0)),
            pl.BlockSpec((NBLK, HEADS * DH), lambda i: (i, 0)),
        ],
        out_shape=[
            jax.ShapeDtypeStruct((N, 4), jnp.float32),
            jax.ShapeDtypeStruct((N, HEADS * DH), jnp.float32),
        ],
    )(x, h, nd1, ndb1, nd2, ndb2, wq)


def _k_edgeattn(nfs_ref, nfd_ref, ce_ref, gq_ref, wk_ref, wv_ref, cd1_ref,
                cdb1_ref, cd2_ref, cdb2_ref, lg_ref, v_ref, cd_ref):
    D = 128 + H
    sf = jnp.concatenate([nfs_ref[:, :D], ce_ref[...]], axis=1)
    k = jnp.dot(sf, wk_ref[...])
    v = jnp.dot(sf, wv_ref[...])
    v_ref[...] = v
    q = gq_ref[:, :HEADS * DH]
    qk = q * k
    blk = qk.shape[0]
    lg_ref[...] = jnp.sum(qk.reshape(blk, HEADS, DH), axis=-1) / np.sqrt(DH)
    ed = jnp.concatenate([nfs_ref[:, :D], nfd_ref[:, :D], ce_ref[...]], axis=1)
    hc = _leaky(jnp.dot(ed, cd1_ref[...]) + cdb1_ref[...])
    cd_ref[...] = jnp.dot(hc, cd2_ref[...]) + cdb2_ref[...]


def _edgeattn(nfs, nfd, ce, gq, wk, wv, cd1, cdb1, cd2, cdb2):
    D = 128 + H
    return pl.pallas_call(
        _k_edgeattn,
        grid=(E // EBLK,),
        in_specs=[
            pl.BlockSpec((EBLK, 256), lambda i: (i, 0)),
            pl.BlockSpec((EBLK, 256), lambda i: (i, 0)),
            pl.BlockSpec((EBLK, 7), lambda i: (i, 0)),
            pl.BlockSpec((EBLK, 128), lambda i: (i, 0)),
            pl.BlockSpec((D + 7, HEADS * DH), lambda i: (0, 0)),
            pl.BlockSpec((D + 7, HEADS * DH), lambda i: (0, 0)),
            pl.BlockSpec((2 * D + 7, H), lambda i: (0, 0)),
            pl.BlockSpec((H,), lambda i: (0,)),
            pl.BlockSpec((H, 4), lambda i: (0, 0)),
            pl.BlockSpec((4,), lambda i: (0,)),
        ],
        out_specs=[
            pl.BlockSpec((EBLK, HEADS), lambda i: (i, 0)),
            pl.BlockSpec((EBLK, HEADS * DH), lambda i: (i, 0)),
            pl.BlockSpec((EBLK, 4), lambda i: (i, 0)),
        ],
        out_shape=[
            jax.ShapeDtypeStruct((E, HEADS), jnp.float32),
            jax.ShapeDtypeStruct((E, HEADS * DH), jnp.float32),
            jax.ShapeDtypeStruct((E, 4), jnp.float32),
        ],
    )(nfs, nfd, ce, gq, wk, wv, cd1, cdb1, cd2, cdb2)


def _k_aohead(x_ref, h_ref, agg_ref, w1_ref, b1_ref, w2_ref, b2_ref, o_ref):
    cat = jnp.concatenate([x_ref[...], h_ref[...], agg_ref[...]], axis=1)
    ha = _leaky(jnp.dot(cat, w1_ref[...]) + b1_ref[...])
    o_ref[...] = jnp.dot(ha, w2_ref[...]) + b2_ref[...]


def _aohead(x, h, agg, w1, b1, w2, b2):
    D = 128 + H
    return pl.pallas_call(
        _k_aohead,
        grid=(N // NBLK,),
        in_specs=[
            pl.BlockSpec((NBLK, 128), lambda i: (i, 0)),
            pl.BlockSpec((NBLK, H), lambda i: (i, 0)),
            pl.BlockSpec((NBLK, HEADS * DH), lambda i: (i, 0)),
            pl.BlockSpec((D + HEADS * DH, H), lambda i: (0, 0)),
            pl.BlockSpec((H,), lambda i: (0,)),
            pl.BlockSpec((H, 8), lambda i: (0, 0)),
            pl.BlockSpec((8,), lambda i: (0,)),
        ],
        out_specs=pl.BlockSpec((NBLK, 8), lambda i: (i, 0)),
        out_shape=jax.ShapeDtypeStruct((N, 8), jnp.float32),
    )(x, h, agg, w1, b1, w2, b2)


def kernel(x, edge_attr, cell_ef, params, edge_index):
    p = params
    src = edge_index[0]
    dst = edge_index[1]

    h = _dimup(x, p['dim_up_w'], p['dim_up_b'])
    for lp in p['nc']:
        gs = _sc_gather(h, src)
        gd = _sc_gather(h, dst)
        m = _msg(gs, gd, edge_attr, lp['msg_w1'], lp['msg_b1'],
                 lp['msg_w2'], lp['msg_b2'])
        gate = jax.nn.sigmoid(m[:, :1])
        f1 = m[:, 1:1 + H1] * gate
        f2 = m[:, 1 + H1:] * gate
        s = jax.ops.segment_sum(f1, dst, num_segments=N)
        mx = _sc_segmax(f2, dst)
        mx = jnp.where(jnp.isfinite(mx), mx, 0.0)
        h = _readout(h, s, mx, lp['ro_w1'], lp['ro_b1'],
                     lp['ro_w2'], lp['ro_b2'])

    net_delays, q_node = _nodehead(x, h, p['nd_w1'], p['nd_b1'],
                                   p['nd_w2'], p['nd_b2'], p['wq'])

    nf1 = jnp.concatenate([x, h], axis=1)
    nfs = _sc_gather(nf1, src)
    nfd = _sc_gather(nf1, dst)
    gq = _sc_gather(q_node, dst)
    logits, v, cell_delays = _edgeattn(nfs, nfd, cell_ef, gq, p['wk'], p['wv'],
                                       p['cd_w1'], p['cd_b1'], p['cd_w2'],
                                       p['cd_b2'])
    lmax = _sc_segmax(logits, dst)
    lmax = jnp.where(jnp.isfinite(lmax), lmax, 0.0)
    ex = jnp.exp(logits - _sc_gather(lmax, dst)[:, :HEADS])
    den = jax.ops.segment_sum(ex, dst, num_segments=N) + 1e-9
    attn = ex / _sc_gather(den, dst)[:, :HEADS]
    w2 = (v.reshape(E, HEADS, DH) * attn[:, :, None]).reshape(E, HEADS * DH)
    agg = jax.ops.segment_sum(w2, dst, num_segments=N)
    atslew = _aohead(x, h, agg, p['ao_w1'], p['ao_b1'], p['ao_w2'], p['ao_b2'])

    return net_delays, cell_delays, atslew


# speedup vs baseline: 5.2427x; 1.2145x over previous
import functools
import numpy as np
import jax
import jax.numpy as jnp
from jax import lax
from jax.experimental import pallas as pl
from jax.experimental.pallas import tpu as pltpu
from jax.experimental.pallas import tpu_sc as plsc

N = 10000
E = 320000
H = 64
H1 = 32
H2 = 32
HEADS = 8
DH = 8

NBLK = 2000
EBLK = 4000


def _leaky(v):
    return jax.nn.leaky_relu(v, 0.2)


_NW = 32


@functools.lru_cache(maxsize=None)
def _make_sc_gather(D, CH):
    per_w = E // _NW
    n_ch = per_w // CH
    mesh = plsc.VectorSubcoreMesh(core_axis_name="c", subcore_axis_name="s")

    @functools.partial(
        pl.kernel,
        mesh=mesh,
        out_type=jax.ShapeDtypeStruct((E, D), jnp.float32),
        scratch_types=[
            pltpu.VMEM((CH,), jnp.int32),
            pltpu.VMEM((CH, D), jnp.float32),
            pltpu.SemaphoreType.DMA,
        ],
    )
    def k(table_hbm, idx_hbm, out_hbm, idx_v, rows_v, sem):
        wid = lax.axis_index("s") * 2 + lax.axis_index("c")
        base = wid * per_w

        def body(i, carry):
            off = base + i * CH
            pltpu.sync_copy(idx_hbm.at[pl.ds(off, CH)], idx_v)
            pltpu.async_copy(table_hbm.at[idx_v], rows_v, sem).wait()
            pltpu.sync_copy(rows_v, out_hbm.at[pl.ds(off, CH)])
            return carry

        lax.fori_loop(0, n_ch, body, 0)

    return k


def _sc_gather(table, idx):
    D = table.shape[1]
    Dp = ((D + 127) // 128) * 128
    if Dp != D:
        table = jnp.pad(table, ((0, 0), (0, Dp - D)))
    return _make_sc_gather(Dp, 400)(table, idx)


@functools.lru_cache(maxsize=None)
def _make_sc_segmax(D, R, CH):
    per_r = E // R
    n_ch = per_r // CH
    n_vec = CH // 16
    rows = N * 8 // 128
    mesh = plsc.VectorSubcoreMesh(core_axis_name="c", subcore_axis_name="s")

    @functools.partial(
        pl.kernel,
        mesh=mesh,
        compiler_params=pltpu.CompilerParams(needs_layout_passes=False),
        out_type=jax.ShapeDtypeStruct((R * D, rows, 128), jnp.float32),
        scratch_types=[
            pltpu.VMEM((rows, 128), jnp.float32),
            pltpu.VMEM((CH,), jnp.int32),
            pltpu.VMEM((CH,), jnp.float32),
        ],
    )
    def k(vals_hbm, idx_hbm, ninf_hbm, out_hbm, acc_v, idx_v, val_v):
        wid = lax.axis_index("s") * 2 + lax.axis_index("c")
        d = wid % D
        r = wid // D
        lane = lax.iota(jnp.int32, 16)
        colv = lax.bitwise_and(lane, 7)
        m_lo = lane < 8
        m_hi = lane >= 8
        pltpu.sync_copy(ninf_hbm, acc_v)

        def chunk_body(c, carry):
            off = r * per_r + c * CH
            pltpu.sync_copy(idx_hbm.at[pl.ds(off, CH)], idx_v)
            pltpu.sync_copy(vals_hbm.at[pl.ds(d * E + off, CH)], val_v)

            def vec_body(j, carry2):
                idxv = idx_v[pl.ds(j * 16, 16)]
                valv = val_v[pl.ds(j * 16, 16)]
                flat = lax.shift_left(idxv, 3) + colv
                rowv = lax.shift_right_logical(flat, 7)
                lanev = lax.bitwise_and(flat, 127)
                g = plsc.load_gather(acc_v, [rowv, lanev])
                plsc.store_scatter(acc_v, [rowv, lanev],
                                   jnp.maximum(g, valv), mask=m_lo)
                g2 = plsc.load_gather(acc_v, [rowv, lanev])
                plsc.store_scatter(acc_v, [rowv, lanev],
                                   jnp.maximum(g2, valv), mask=m_hi)
                return carry2

            lax.fori_loop(0, n_vec, vec_body, 0)
            return carry

        lax.fori_loop(0, n_ch, chunk_body, 0)
        pltpu.sync_copy(acc_v, out_hbm.at[r * D + d])

    return k


_NEG_INF_ACC = None


def _sc_segmax(vals, dst):
    D = vals.shape[1]
    R = _NW // D
    vals_flat = vals.T.reshape(-1)
    ninf = jnp.full((N * 8 // 128, 128), -jnp.inf, jnp.float32)
    out = _make_sc_segmax(D, R, 4000)(vals_flat, dst, ninf)
    part = out.reshape(R, D, N, 8)
    return jnp.max(part, axis=(0, 3)).T


@functools.lru_cache(maxsize=None)
def _make_sc_segsum(D, CH):
    per_w = E // _NW
    n_ch = per_w // CH
    mesh = plsc.VectorSubcoreMesh(core_axis_name="c", subcore_axis_name="s")

    @functools.partial(
        pl.kernel,
        mesh=mesh,
        out_type=jax.ShapeDtypeStruct((2, N, D), jnp.float32),
        scratch_types=[
            pltpu.VMEM_SHARED((N, D), jnp.float32),
            pltpu.VMEM((CH,), jnp.int32),
            pltpu.VMEM((CH, D), jnp.float32),
        ],
    )
    def k(vals_hbm, idx_hbm, zeros_hbm, out_hbm, acc_sh, idx_v, val_v):
        sid = lax.axis_index("s")
        cid = lax.axis_index("c")
        wid = sid * 2 + cid

        @pl.when(sid == 0)
        def _init():
            pltpu.sync_copy(zeros_hbm, acc_sh)

        plsc.subcore_barrier()

        def chunk_body(c, carry):
            off = wid * per_w + c * CH
            pltpu.sync_copy(idx_hbm.at[pl.ds(off, CH)], idx_v)
            pltpu.sync_copy(vals_hbm.at[pl.ds(off, CH)], val_v)
            pltpu.sync_copy(val_v, acc_sh.at[idx_v], add=True)
            return carry

        lax.fori_loop(0, n_ch, chunk_body, 0)
        plsc.subcore_barrier()

        @pl.when(sid == 0)
        def _out():
            pltpu.sync_copy(acc_sh, out_hbm.at[cid])

    return k


def _sc_segsum(vals, dst):
    D = vals.shape[1]
    if D != 128:
        vals = jnp.pad(vals, ((0, 0), (0, 128 - D)))
    zeros = jnp.zeros((N, 128), jnp.float32)
    out = _make_sc_segsum(128, 200)(vals, dst, zeros)
    return (out[0] + out[1])[:, :D]


def _k_dimup(x_ref, w_ref, b_ref, o_ref):
    o_ref[...] = _leaky(jnp.dot(x_ref[...], w_ref[...]) + b_ref[...])


def _dimup(x, w, b):
    return pl.pallas_call(
        _k_dimup,
        grid=(N // NBLK,),
        in_specs=[
            pl.BlockSpec((NBLK, 128), lambda i: (i, 0)),
            pl.BlockSpec((128, H), lambda i: (0, 0)),
            pl.BlockSpec((H,), lambda i: (0,)),
        ],
        out_specs=pl.BlockSpec((NBLK, H), lambda i: (i, 0)),
        out_shape=jax.ShapeDtypeStruct((N, H), jnp.float32),
    )(x, w, b)


def _k_msg(gs_ref, gd_ref, ea_ref, w1_ref, b1_ref, w2_ref, b2_ref, m_ref):
    m_in = jnp.concatenate([gs_ref[:, :H], gd_ref[:, :H], ea_ref[...]], axis=1)
    hm = _leaky(jnp.dot(m_in, w1_ref[...]) + b1_ref[...])
    m_ref[...] = jnp.dot(hm, w2_ref[...]) + b2_ref[...]


def _msg(gs, gd, ea, w1, b1, w2, b2):
    return pl.pallas_call(
        _k_msg,
        grid=(E // EBLK,),
        in_specs=[
            pl.BlockSpec((EBLK, 128), lambda i: (i, 0)),
            pl.BlockSpec((EBLK, 128), lambda i: (i, 0)),
            pl.BlockSpec((EBLK, 2), lambda i: (i, 0)),
            pl.BlockSpec((2 * H + 2, H), lambda i: (0, 0)),
            pl.BlockSpec((H,), lambda i: (0,)),
            pl.BlockSpec((H, 1 + H1 + H2), lambda i: (0, 0)),
            pl.BlockSpec((1 + H1 + H2,), lambda i: (0,)),
        ],
        out_specs=pl.BlockSpec((EBLK, 1 + H1 + H2), lambda i: (i, 0)),
        out_shape=jax.ShapeDtypeStruct((E, 1 + H1 + H2), jnp.float32),
    )(gs, gd, ea, w1, b1, w2, b2)


def _k_readout(h_ref, s_ref, mx_ref, w1_ref, b1_ref, w2_ref, b2_ref, o_ref):
    r = jnp.concatenate([h_ref[...], s_ref[...], mx_ref[...]], axis=1)
    hr = _leaky(jnp.dot(r, w1_ref[...]) + b1_ref[...])
    o_ref[...] = jnp.dot(hr, w2_ref[...]) + b2_ref[...] + h_ref[...]


def _readout(h, s, mx, w1, b1, w2, b2):
    return pl.pallas_call(
        _k_readout,
        grid=(N // NBLK,),
        in_specs=[
            pl.BlockSpec((NBLK, H), lambda i: (i, 0)),
            pl.BlockSpec((NBLK, H1), lambda i: (i, 0)),
            pl.BlockSpec((NBLK, H2), lambda i: (i, 0)),
            pl.BlockSpec((H + H1 + H2, H), lambda i: (0, 0)),
            pl.BlockSpec((H,), lambda i: (0,)),
            pl.BlockSpec((H, H), lambda i: (0, 0)),
            pl.BlockSpec((H,), lambda i: (0,)),
        ],
        out_specs=pl.BlockSpec((NBLK, H), lambda i: (i, 0)),
        out_shape=jax.ShapeDtypeStruct((N, H), jnp.float32),
    )(h, s, mx, w1, b1, w2, b2)


def _k_nodehead(x_ref, h_ref, nd1_ref, ndb1_ref, nd2_ref, ndb2_ref, wq_ref,
                nd_ref, q_ref):
    hn = _leaky(jnp.dot(h_ref[...], nd1_ref[...]) + ndb1_ref[...])
    nd_ref[...] = jnp.dot(hn, nd2_ref[...]) + ndb2_ref[...]
    nf1 = jnp.concatenate([x_ref[...], h_ref[...]], axis=1)
    q_ref[...] = jnp.dot(nf1, wq_ref[...])


def _nodehead(x, h, nd1, ndb1, nd2, ndb2, wq):
    return pl.pallas_call(
        _k_nodehead,
        grid=(N // NBLK,),
        in_specs=[
            pl.BlockSpec((NBLK, 128), lambda i: (i, 0)),
            pl.BlockSpec((NBLK, H), lambda i: (i, 0)),
            pl.BlockSpec((H, H), lambda i: (0, 0)),
            pl.BlockSpec((H,), lambda i: (0,)),
            pl.BlockSpec((H, 4), lambda i: (0, 0)),
            pl.BlockSpec((4,), lambda i: (0,)),
            pl.BlockSpec((128 + H, HEADS * DH), lambda i: (0, 0)),
        ],
        out_specs=[
            pl.BlockSpec((NBLK, 4), lambda i: (i, 0)),
            pl.BlockSpec((NBLK, HEADS * DH), lambda i: (i, 0)),
        ],
        out_shape=[
            jax.ShapeDtypeStruct((N, 4), jnp.float32),
            jax.ShapeDtypeStruct((N, HEADS * DH), jnp.float32),
        ],
    )(x, h, nd1, ndb1, nd2, ndb2, wq)


def _k_edgeattn(nfs_ref, nfd_ref, ce_ref, gq_ref, wk_ref, wv_ref, cd1_ref,
                cdb1_ref, cd2_ref, cdb2_ref, lg_ref, v_ref, cd_ref):
    D = 128 + H
    sf = jnp.concatenate([nfs_ref[:, :D], ce_ref[...]], axis=1)
    k = jnp.dot(sf, wk_ref[...])
    v = jnp.dot(sf, wv_ref[...])
    v_ref[...] = v
    q = gq_ref[:, :HEADS * DH]
    qk = q * k
    blk = qk.shape[0]
    lg_ref[...] = jnp.sum(qk.reshape(blk, HEADS, DH), axis=-1) / np.sqrt(DH)
    ed = jnp.concatenate([nfs_ref[:, :D], nfd_ref[:, :D], ce_ref[...]], axis=1)
    hc = _leaky(jnp.dot(ed, cd1_ref[...]) + cdb1_ref[...])
    cd_ref[...] = jnp.dot(hc, cd2_ref[...]) + cdb2_ref[...]


def _edgeattn(nfs, nfd, ce, gq, wk, wv, cd1, cdb1, cd2, cdb2):
    D = 128 + H
    return pl.pallas_call(
        _k_edgeattn,
        grid=(E // EBLK,),
        in_specs=[
            pl.BlockSpec((EBLK, 256), lambda i: (i, 0)),
            pl.BlockSpec((EBLK, 256), lambda i: (i, 0)),
            pl.BlockSpec((EBLK, 7), lambda i: (i, 0)),
            pl.BlockSpec((EBLK, 128), lambda i: (i, 0)),
            pl.BlockSpec((D + 7, HEADS * DH), lambda i: (0, 0)),
            pl.BlockSpec((D + 7, HEADS * DH), lambda i: (0, 0)),
            pl.BlockSpec((2 * D + 7, H), lambda i: (0, 0)),
            pl.BlockSpec((H,), lambda i: (0,)),
            pl.BlockSpec((H, 4), lambda i: (0, 0)),
            pl.BlockSpec((4,), lambda i: (0,)),
        ],
        out_specs=[
            pl.BlockSpec((EBLK, HEADS), lambda i: (i, 0)),
            pl.BlockSpec((EBLK, HEADS * DH), lambda i: (i, 0)),
            pl.BlockSpec((EBLK, 4), lambda i: (i, 0)),
        ],
        out_shape=[
            jax.ShapeDtypeStruct((E, HEADS), jnp.float32),
            jax.ShapeDtypeStruct((E, HEADS * DH), jnp.float32),
            jax.ShapeDtypeStruct((E, 4), jnp.float32),
        ],
    )(nfs, nfd, ce, gq, wk, wv, cd1, cdb1, cd2, cdb2)


def _k_aohead(x_ref, h_ref, agg_ref, w1_ref, b1_ref, w2_ref, b2_ref, o_ref):
    cat = jnp.concatenate([x_ref[...], h_ref[...], agg_ref[...]], axis=1)
    ha = _leaky(jnp.dot(cat, w1_ref[...]) + b1_ref[...])
    o_ref[...] = jnp.dot(ha, w2_ref[...]) + b2_ref[...]


def _aohead(x, h, agg, w1, b1, w2, b2):
    D = 128 + H
    return pl.pallas_call(
        _k_aohead,
        grid=(N // NBLK,),
        in_specs=[
            pl.BlockSpec((NBLK, 128), lambda i: (i, 0)),
            pl.BlockSpec((NBLK, H), lambda i: (i, 0)),
            pl.BlockSpec((NBLK, HEADS * DH), lambda i: (i, 0)),
            pl.BlockSpec((D + HEADS * DH, H), lambda i: (0, 0)),
            pl.BlockSpec((H,), lambda i: (0,)),
            pl.BlockSpec((H, 8), lambda i: (0, 0)),
            pl.BlockSpec((8,), lambda i: (0,)),
        ],
        out_specs=pl.BlockSpec((NBLK, 8), lambda i: (i, 0)),
        out_shape=jax.ShapeDtypeStruct((N, 8), jnp.float32),
    )(x, h, agg, w1, b1, w2, b2)


def kernel(x, edge_attr, cell_ef, params, edge_index):
    p = params
    src = edge_index[0]
    dst = edge_index[1]

    h = _dimup(x, p['dim_up_w'], p['dim_up_b'])
    for lp in p['nc']:
        gs = _sc_gather(h, src)
        gd = _sc_gather(h, dst)
        m = _msg(gs, gd, edge_attr, lp['msg_w1'], lp['msg_b1'],
                 lp['msg_w2'], lp['msg_b2'])
        gate = jax.nn.sigmoid(m[:, :1])
        f1 = m[:, 1:1 + H1] * gate
        f2 = m[:, 1 + H1:] * gate
        s = _sc_segsum(f1, dst)
        mx = _sc_segmax(f2, dst)
        mx = jnp.where(jnp.isfinite(mx), mx, 0.0)
        h = _readout(h, s, mx, lp['ro_w1'], lp['ro_b1'],
                     lp['ro_w2'], lp['ro_b2'])

    net_delays, q_node = _nodehead(x, h, p['nd_w1'], p['nd_b1'],
                                   p['nd_w2'], p['nd_b2'], p['wq'])

    nf1 = jnp.concatenate([x, h], axis=1)
    nfs = _sc_gather(nf1, src)
    nfd = _sc_gather(nf1, dst)
    gq = _sc_gather(q_node, dst)
    logits, v, cell_delays = _edgeattn(nfs, nfd, cell_ef, gq, p['wk'], p['wv'],
                                       p['cd_w1'], p['cd_b1'], p['cd_w2'],
                                       p['cd_b2'])
    lmax = _sc_segmax(logits, dst)
    lmax = jnp.where(jnp.isfinite(lmax), lmax, 0.0)
    ex = jnp.exp(logits - _sc_gather(lmax, dst)[:, :HEADS])
    den = _sc_segsum(ex, dst) + 1e-9
    attn = ex / _sc_gather(den, dst)[:, :HEADS]
    w2 = (v.reshape(E, HEADS, DH) * attn[:, :, None]).reshape(E, HEADS * DH)
    agg = _sc_segsum(w2, dst)
    atslew = _aohead(x, h, agg, p['ao_w1'], p['ao_b1'], p['ao_w2'], p['ao_b2'])

    return net_delays, cell_delays, atslew
